# speedup vs baseline: 5.7995x; 5.7995x over previous
import functools

import jax
import jax.numpy as jnp
from jax import lax
from jax.experimental import pallas as pl
from jax.experimental.pallas import tpu as pltpu
from jax.experimental.pallas import tpu_sc as plsc

_info = plsc.get_sparse_core_info()
_NC, _NS = _info.num_cores, _info.num_subcores
_NW = _NC * _NS

_CH = 128
_NBUF = 2


def _gather_rows(ids_flat, table):
    B = ids_flat.shape[0]
    D = table.shape[1]
    assert B % (_NW * _CH * _NBUF) == 0
    b_per_w = B // _NW
    nchunk = b_per_w // _CH
    nround = nchunk // _NBUF

    mesh = plsc.VectorSubcoreMesh(core_axis_name="c", subcore_axis_name="s")

    @functools.partial(
        pl.kernel,
        out_type=jax.ShapeDtypeStruct((B, D), jnp.float32),
        mesh=mesh,
        scratch_types=[
            pltpu.VMEM((b_per_w,), jnp.int32),
            pltpu.VMEM((_CH, D), jnp.float32),
            pltpu.VMEM((_CH, D), jnp.float32),
            pltpu.SemaphoreType.DMA,
            pltpu.SemaphoreType.DMA,
            pltpu.SemaphoreType.DMA,
            pltpu.SemaphoreType.DMA,
        ],
    )
    def sc_kernel(table_hbm, ids_hbm, out_hbm, idx_v, buf0, buf1,
                  g0, g1, s0, s1):
        wid = lax.axis_index("s") * _NC + lax.axis_index("c")
        base = wid * b_per_w
        pltpu.sync_copy(ids_hbm.at[pl.ds(base, b_per_w)], idx_v)

        bufs = (buf0, buf1)
        gsem = (g0, g1)
        ssem = (s0, s1)

        def g_start(i, b):
            pltpu.async_copy(
                table_hbm.at[idx_v.at[pl.ds(i * _CH, _CH)]], bufs[b], gsem[b])

        def g_wait(b):
            pltpu.make_async_copy(
                table_hbm.at[idx_v.at[pl.ds(0, _CH)]], bufs[b], gsem[b]).wait()

        def s_start(i, b):
            pltpu.async_copy(
                bufs[b], out_hbm.at[pl.ds(base + i * _CH, _CH)], ssem[b])

        def s_wait(b):
            pltpu.make_async_copy(
                bufs[b], out_hbm.at[pl.ds(base, _CH)], ssem[b]).wait()

        for b in range(_NBUF):
            g_start(b, b)

        def round_body(j, carry):
            for b in range(_NBUF):
                i = j * _NBUF + b
                g_wait(b)
                s_start(i, b)
            for b in range(_NBUF):
                s_wait(b)
                g_start((j + 1) * _NBUF + b, b)
            return carry

        lax.fori_loop(0, nround - 1, round_body, 0)

        for b in range(_NBUF):
            i = (nround - 1) * _NBUF + b
            g_wait(b)
            s_start(i, b)
        for b in range(_NBUF):
            s_wait(b)

    return sc_kernel(table, ids_flat)


def kernel(input_ids, table):
    ids_flat = input_ids.reshape(-1).astype(jnp.int32)
    out = _gather_rows(ids_flat, table)
    return out.reshape(input_ids.shape + (table.shape[1],))


# speedup vs baseline: 5.8148x; 1.0026x over previous
import functools

import jax
import jax.numpy as jnp
from jax import lax
from jax.experimental import pallas as pl
from jax.experimental.pallas import tpu as pltpu
from jax.experimental.pallas import tpu_sc as plsc

_info = plsc.get_sparse_core_info()
_NC, _NS = _info.num_cores, _info.num_subcores
_NW = _NC * _NS

_CH = 80
_NBUF = 4


def _gather_rows(ids_flat, table):
    B = ids_flat.shape[0]
    D = table.shape[1]
    assert B % (_NW * _CH * _NBUF) == 0
    b_per_w = B // _NW
    nchunk = b_per_w // _CH
    nround = nchunk // _NBUF

    mesh = plsc.VectorSubcoreMesh(core_axis_name="c", subcore_axis_name="s")

    @functools.partial(
        pl.kernel,
        out_type=jax.ShapeDtypeStruct((B, D), jnp.float32),
        mesh=mesh,
        scratch_types=(
            [pltpu.VMEM((b_per_w,), jnp.int32)]
            + [pltpu.VMEM((_CH, D), jnp.float32)] * _NBUF
            + [pltpu.SemaphoreType.DMA] * (2 * _NBUF)
        ),
    )
    def sc_kernel(table_hbm, ids_hbm, out_hbm, idx_v, *bufs_and_sems):
        wid = lax.axis_index("s") * _NC + lax.axis_index("c")
        base = wid * b_per_w
        pltpu.sync_copy(ids_hbm.at[pl.ds(base, b_per_w)], idx_v)

        bufs = bufs_and_sems[:_NBUF]
        gsem = bufs_and_sems[_NBUF:2 * _NBUF]
        ssem = bufs_and_sems[2 * _NBUF:]

        def g_start(i, b):
            pltpu.async_copy(
                table_hbm.at[idx_v.at[pl.ds(i * _CH, _CH)]], bufs[b], gsem[b])

        def g_wait(b):
            pltpu.make_async_copy(
                table_hbm.at[idx_v.at[pl.ds(0, _CH)]], bufs[b], gsem[b]).wait()

        def s_start(i, b):
            pltpu.async_copy(
                bufs[b], out_hbm.at[pl.ds(base + i * _CH, _CH)], ssem[b])

        def s_wait(b):
            pltpu.make_async_copy(
                bufs[b], out_hbm.at[pl.ds(base, _CH)], ssem[b]).wait()

        for b in range(_NBUF):
            g_start(b, b)

        def round_body(j, carry):
            for b in range(_NBUF):
                i = j * _NBUF + b
                g_wait(b)
                s_start(i, b)
            for b in range(_NBUF):
                s_wait(b)
                g_start((j + 1) * _NBUF + b, b)
            return carry

        lax.fori_loop(0, nround - 1, round_body, 0)

        for b in range(_NBUF):
            i = (nround - 1) * _NBUF + b
            g_wait(b)
            s_start(i, b)
        for b in range(_NBUF):
            s_wait(b)

    return sc_kernel(table, ids_flat)


def kernel(input_ids, table):
    ids_flat = input_ids.reshape(-1).astype(jnp.int32)
    out = _gather_rows(ids_flat, table)
    return out.reshape(input_ids.shape + (table.shape[1],))
